# Initial kernel scaffold; baseline (speedup 1.0000x reference)
import jax
import jax.numpy as jnp
from jax.experimental import pallas as pl
from jax.experimental.pallas import tpu as pltpu
from jax.experimental.pallas import tpu_sc as plsc

_D = 64
_W = 512


def kernel(token_ids, weight):
    b, s = token_ids.shape
    n = b * s
    idx = token_ids.reshape(1, n).astype(jnp.int32)
    mesh = plsc.VectorSubcoreMesh(core_axis_name="c", subcore_axis_name="s")

    @jax.jit
    def run(weight, idx):
        @pl.kernel(
            out_type=jax.ShapeDtypeStruct((n, _D), weight.dtype),
            mesh=mesh,
        )
        def k(w_hbm, i_hbm, o_hbm):
            def body(i_vmem, o_vmem):
                pltpu.sync_copy(w_hbm.at[i_vmem.at[0]], o_vmem)

            pltpu.emit_pipeline(
                body,
                grid=(n // _W,),
                in_specs=[pl.BlockSpec((1, _W), index_map=lambda i: (0, i))],
                out_specs=[pl.BlockSpec((_W, _D), index_map=lambda i: (i, 0))],
                core_axis_name=("c", "s"),
                dimension_semantics=(pltpu.PARALLEL,),
            )(i_hbm, o_hbm)

        return k(weight, idx)

    return run(weight, idx).reshape(b, s, _D)


# speedup vs baseline: 1.1180x; 1.1180x over previous
import jax
import jax.numpy as jnp
from jax import lax
from jax.experimental import pallas as pl
from jax.experimental.pallas import tpu as pltpu
from jax.experimental.pallas import tpu_sc as plsc

_D = 64
_SB = 512
_NG = _SB // 128


def kernel(token_ids, weight):
    b, s = token_ids.shape
    v, d = weight.shape
    tokT = token_ids.T.astype(jnp.int32)
    wpk = weight.reshape(v // 2, 2 * d)
    mesh = plsc.VectorSubcoreMesh(core_axis_name="c", subcore_axis_name="s")

    @pl.kernel(
        out_type=jax.ShapeDtypeStruct((s, d, b), jnp.float32),
        mesh=mesh,
        scratch_types=[
            pltpu.VMEM((_SB,), jnp.int32),
            pltpu.VMEM((_NG, 128), jnp.int32),
            pltpu.VMEM((_NG, 128, 2 * _D), jnp.float32),
            pltpu.VMEM((_D, _SB), jnp.float32),
            pltpu.SemaphoreType.DMA,
        ],
        compiler_params=pltpu.CompilerParams(needs_layout_passes=False),
    )
    def gather_kernel(t_hbm, w_hbm, o_hbm, idx_v, q_v, rows_v, plane_v, sem):
        wid = lax.axis_index("s") * 2 + lax.axis_index("c")
        s0 = wid * _SB
        lane = lax.iota(jnp.int32, 16)

        def pbody(p, carry):
            pltpu.sync_copy(t_hbm.at[p, pl.ds(s0, _SB)], idx_v)
            for g in range(_SB // 16):
                vi = idx_v[pl.ds(g * 16, 16)]
                q_v[g // 8, pl.ds((g % 8) * 16, 16)] = lax.shift_right_logical(vi, 1)
            copies = [
                pltpu.async_copy(w_hbm.at[q_v.at[j]], rows_v.at[j], sem)
                for j in range(_NG)
            ]
            for c in copies:
                c.wait()

            def gbody(g, carry2):
                t = g * 16 + lane
                vi = idx_v[pl.ds(g * 16, 16)]
                j_vec = lax.shift_right_logical(t, 7)
                r_vec = lax.bitwise_and(t, 127)
                base = lax.bitwise_and(vi, 1) * _D
                for dd in range(_D):
                    vals = plsc.load_gather(rows_v, [j_vec, r_vec, base + dd])
                    plane_v[dd, pl.ds(g * 16, 16)] = vals
                return carry2

            lax.fori_loop(0, _SB // 16, gbody, 0)
            pltpu.sync_copy(plane_v, o_hbm.at[p, :, pl.ds(s0, _SB)])
            return carry

        lax.fori_loop(0, s, pbody, 0)

    outT = gather_kernel(tokT, wpk)
    return outT.transpose(2, 0, 1)


# speedup vs baseline: 1.2582x; 1.1254x over previous
import jax
import jax.numpy as jnp
from jax import lax
from jax.experimental import pallas as pl
from jax.experimental.pallas import tpu as pltpu
from jax.experimental.pallas import tpu_sc as plsc

_D = 64
_SB = 512
_CH = 128


def kernel(token_ids, weight):
    b, s = token_ids.shape
    v, d = weight.shape
    tokT = token_ids.T.astype(jnp.int32)
    wpk = weight.reshape(v // 2, 2 * d)
    mesh = plsc.VectorSubcoreMesh(core_axis_name="c", subcore_axis_name="s")
    nsteps = s * (_SB // _CH)

    @pl.kernel(
        out_type=jax.ShapeDtypeStruct((s, d, b), jnp.float32),
        mesh=mesh,
        scratch_types=[
            pltpu.VMEM((_SB,), jnp.int32),
            pltpu.VMEM((nsteps * _CH,), jnp.int32),
            pltpu.VMEM((nsteps * _CH,), jnp.int32),
            pltpu.VMEM((2, _CH, 2 * _D), jnp.float32),
            pltpu.VMEM((2, _D, _CH), jnp.float32),
            pltpu.SemaphoreType.DMA,
            pltpu.SemaphoreType.DMA,
            pltpu.SemaphoreType.DMA,
            pltpu.SemaphoreType.DMA,
        ],
        compiler_params=pltpu.CompilerParams(needs_layout_passes=False),
    )
    def gather_kernel(t_hbm, w_hbm, o_hbm, idx_v, q_v, base_v, rows_v,
                      plane_v, sg0, sg1, so0, so1):
        wid = lax.axis_index("s") * 2 + lax.axis_index("c")
        s0 = wid * _SB
        lane = lax.iota(jnp.int32, 16)

        def pstage(p, carry):
            pltpu.sync_copy(t_hbm.at[p, pl.ds(s0, _SB)], idx_v)

            def gstage(u, c2):
                vi = idx_v[pl.ds(u * 16, 16)]
                off = p * _SB + u * 16
                q_v[pl.ds(off, 16)] = lax.shift_right_logical(vi, 1)
                base_v[pl.ds(off, 16)] = lax.bitwise_and(vi, 1) * _D
                return c2

            lax.fori_loop(0, _SB // 16, gstage, 0)
            return carry

        lax.fori_loop(0, s, pstage, 0)

        def issue(k, buf, sem):
            pltpu.async_copy(
                w_hbm.at[q_v.at[pl.ds(k * _CH, _CH)]], rows_v.at[buf], sem)

        def wait_gather(buf, sem):
            pltpu.make_async_copy(
                w_hbm.at[q_v.at[pl.ds(0, _CH)]], rows_v.at[buf], sem).wait()

        def wait_out(buf, sem):
            pltpu.make_async_copy(
                plane_v.at[buf], o_hbm.at[0, :, pl.ds(s0, _CH)], sem).wait()

        def compute(k, buf):
            rows2 = rows_v.at[buf]
            plane2 = plane_v.at[buf]

            def gbody(u, c2):
                r_vec = u * 16 + lane
                bb = base_v[pl.ds(k * _CH + u * 16, 16)]
                for dd in range(_D):
                    vals = plsc.load_gather(rows2, [r_vec, bb + dd])
                    plane2[dd, pl.ds(u * 16, 16)] = vals
                return c2

            lax.fori_loop(0, _CH // 16, gbody, 0)

        def start_out(k, buf, sem):
            p = lax.shift_right_logical(k, 2)
            g = lax.bitwise_and(k, 3)
            pltpu.async_copy(
                plane_v.at[buf], o_hbm.at[p, :, pl.ds(s0 + g * _CH, _CH)],
                sem)

        issue(0, 0, sg0)
        issue(1, 1, sg1)

        def body(i, carry):
            k0 = i * 2
            k1 = k0 + 1

            @pl.when(i > 0)
            def _():
                wait_out(0, so0)

            wait_gather(0, sg0)
            compute(k0, 0)

            @pl.when(i < nsteps // 2 - 1)
            def _():
                issue(k0 + 2, 0, sg0)

            start_out(k0, 0, so0)

            @pl.when(i > 0)
            def _():
                wait_out(1, so1)

            wait_gather(1, sg1)
            compute(k1, 1)

            @pl.when(i < nsteps // 2 - 1)
            def _():
                issue(k1 + 2, 1, sg1)

            start_out(k1, 1, so1)
            return carry

        lax.fori_loop(0, nsteps // 2, body, 0)
        wait_out(0, so0)
        wait_out(1, so1)

    outT = gather_kernel(tokT, wpk)
    return outT.transpose(2, 0, 1)


# speedup vs baseline: 1.2583x; 1.0001x over previous
import jax
import jax.numpy as jnp
from jax import lax
from jax.experimental import pallas as pl
from jax.experimental.pallas import tpu as pltpu
from jax.experimental.pallas import tpu_sc as plsc

_D = 64
_SB = 512
_CH = 128


def kernel(token_ids, weight):
    b, s = token_ids.shape
    v, d = weight.shape
    tokT = token_ids.T.astype(jnp.int32)
    wpk = weight.reshape(v // 2, 2 * d)
    mesh = plsc.VectorSubcoreMesh(core_axis_name="c", subcore_axis_name="s")
    nsteps = s * (_SB // _CH)

    @pl.kernel(
        out_type=jax.ShapeDtypeStruct((s, d, b), jnp.float32),
        mesh=mesh,
        scratch_types=[
            pltpu.VMEM((_SB,), jnp.int32),
            pltpu.VMEM((nsteps * _CH,), jnp.int32),
            pltpu.VMEM((nsteps * _CH,), jnp.int32),
            pltpu.VMEM((2, _CH, 2 * _D), jnp.float32),
            pltpu.VMEM((2, _D, _CH), jnp.float32),
            pltpu.SemaphoreType.DMA,
            pltpu.SemaphoreType.DMA,
            pltpu.SemaphoreType.DMA,
            pltpu.SemaphoreType.DMA,
        ],
        compiler_params=pltpu.CompilerParams(needs_layout_passes=False),
    )
    def gather_kernel(t_hbm, w_hbm, o_hbm, idx_v, q_v, base_v, rows_v,
                      plane_v, sg0, sg1, so0, so1):
        wid = lax.axis_index("s") * 2 + lax.axis_index("c")
        s0 = wid * _SB
        lane = lax.iota(jnp.int32, 16)

        def pstage(p, carry):
            pltpu.sync_copy(t_hbm.at[p, pl.ds(s0, _SB)], idx_v)

            def gstage(u, c2):
                vi = idx_v[pl.ds(u * 16, 16)]
                off = p * _SB + u * 16
                q_v[pl.ds(off, 16)] = lax.shift_right_logical(vi, 1)
                base_v[pl.ds(off, 16)] = lax.bitwise_and(vi, 1) * _D
                return c2

            lax.fori_loop(0, _SB // 16, gstage, 0)
            return carry

        lax.fori_loop(0, s, pstage, 0)

        def issue(k, buf, sem):
            pltpu.async_copy(
                w_hbm.at[q_v.at[pl.ds(k * _CH, _CH)]], rows_v.at[buf], sem)

        def wait_gather(buf, sem):
            pltpu.make_async_copy(
                w_hbm.at[q_v.at[pl.ds(0, _CH)]], rows_v.at[buf], sem).wait()

        def wait_out(buf, sem):
            pltpu.make_async_copy(
                plane_v.at[buf], o_hbm.at[0, :, pl.ds(s0, _CH)], sem).wait()

        def compute(k, buf):
            plane2 = plane_v.at[buf]

            def gbody(u, c2):
                r_vec = u * 16 + lane
                bb = base_v[pl.ds(k * _CH + u * 16, 16)]
                for dd in range(_D):
                    vals = plsc.load_gather(rows_v.at[buf], [r_vec, bb + dd])
                    plane2[dd, pl.ds(u * 16, 16)] = vals
                return c2

            lax.fori_loop(0, _CH // 16, gbody, 0)

        def start_out(k, buf, sem):
            p = lax.shift_right_logical(k, 2)
            g = lax.bitwise_and(k, 3)
            pltpu.async_copy(
                plane_v.at[buf], o_hbm.at[p, :, pl.ds(s0 + g * _CH, _CH)],
                sem)

        issue(0, 0, sg0)
        issue(1, 1, sg1)

        def body(i, carry):
            k0 = i * 2
            k1 = k0 + 1

            @pl.when(i > 0)
            def _():
                wait_out(0, so0)

            wait_gather(0, sg0)
            compute(k0, 0)

            @pl.when(i < nsteps // 2 - 1)
            def _():
                issue(k0 + 2, 0, sg0)

            start_out(k0, 0, so0)

            @pl.when(i > 0)
            def _():
                wait_out(1, so1)

            wait_gather(1, sg1)
            compute(k1, 1)

            @pl.when(i < nsteps // 2 - 1)
            def _():
                issue(k1 + 2, 1, sg1)

            start_out(k1, 1, so1)
            return carry

        lax.fori_loop(0, nsteps // 2, body, 0)
        wait_out(0, so0)
        wait_out(1, so1)

    outT = gather_kernel(tokT, wpk)
    return outT.transpose(2, 0, 1)
